# baseline (speedup 1.0000x reference)
import functools
import math

import jax
import jax.numpy as jnp
from jax import lax
from jax.experimental import pallas as pl
from jax.experimental.pallas import tpu as pltpu
from jax.experimental.pallas import tpu_sc as plsc

NC = 2
NS = 16
L = 16
NW = NC * NS
CHUNK = 128
NBUF = 4


@functools.lru_cache(maxsize=None)
def _build(total_rows: int, vocab: int, d: int):
    assert d % L == 0
    chunks_total = total_rows // CHUNK
    assert chunks_total * CHUNK == total_rows and chunks_total % NW == 0
    chunks_per_w = chunks_total // NW
    rows_per_w = chunks_per_w * CHUNK
    scale = math.sqrt(d)
    mesh = plsc.VectorSubcoreMesh(
        core_axis_name="c", subcore_axis_name="s",
        num_cores=NC, num_subcores=NS)

    def body(idx_hbm, table_hbm, out_hbm, idx_v, rows_v, *sems):
        gsems = sems[:NBUF]
        ssems = sems[NBUF:]
        wid = lax.axis_index("s") * NC + lax.axis_index("c")
        c0 = wid * chunks_per_w
        r0 = wid * rows_per_w

        pltpu.sync_copy(idx_hbm.at[pl.ds(c0, chunks_per_w)], idx_v)

        def gather(c, b):
            return pltpu.make_async_copy(
                table_hbm.at[idx_v.at[c]], rows_v.at[b], gsems[b])

        def scatter(c, b):
            return pltpu.make_async_copy(
                rows_v.at[b], out_hbm.at[pl.ds(r0 + c * CHUNK, CHUNK)],
                ssems[b])

        gather(0, 0).start()
        gather(1, 1).start()

        @pl.loop(0, chunks_per_w, step=NBUF)
        def _group(g):
            for b in range(NBUF):
                c = g + b
                if b >= 2:
                    scatter(c - 2, (b + 2) % NBUF).wait()
                else:
                    @pl.when(c >= 2)
                    def _():
                        scatter(c - 2, (b + 2) % NBUF).wait()
                @pl.when(c + 2 < chunks_per_w)
                def _():
                    gather(c + 2, (b + 2) % NBUF).start()
                gather(c, b).wait()
                buf = rows_v.at[b]

                @pl.loop(0, CHUNK, unroll=8)
                def _row(j):
                    for k in range(d // L):
                        sl = pl.ds(k * L, L)
                        buf[j, sl] = buf[j, sl] * scale

                scatter(c, b).start()

        scatter(chunks_per_w - 2, (chunks_per_w - 2) % NBUF).wait()
        scatter(chunks_per_w - 1, (chunks_per_w - 1) % NBUF).wait()

    return pl.kernel(
        body,
        out_type=jax.ShapeDtypeStruct((total_rows, d), jnp.float32),
        mesh=mesh,
        scratch_types=[
            pltpu.VMEM((chunks_per_w, CHUNK), jnp.int32),
            pltpu.VMEM((NBUF, CHUNK, d), jnp.float32),
        ] + [pltpu.SemaphoreType.DMA] * (2 * NBUF),
        compiler_params=pltpu.CompilerParams(use_tc_tiling_on_sc=False),
    )


def kernel(x, table):
    b, s = x.shape
    vocab, d = table.shape
    total = b * s
    idx = x.astype(jnp.int32).reshape(total // CHUNK, CHUNK)
    out = _build(total, vocab, d)(idx, table)
    return out.reshape(b, s, d)


# speedup vs baseline: 1.0012x; 1.0012x over previous
import functools
import math

import jax
import jax.numpy as jnp
from jax import lax
from jax.experimental import pallas as pl
from jax.experimental.pallas import tpu as pltpu
from jax.experimental.pallas import tpu_sc as plsc

NC = 2
NS = 16
L = 16
NW = NC * NS
NBUF = 4


@functools.lru_cache(maxsize=None)
def _build(b: int, s: int, vocab: int, d: int):
    assert d % L == 0 and b % NW == 0
    rows_per_w = b // NW
    scale = math.sqrt(d)
    mesh = plsc.VectorSubcoreMesh(
        core_axis_name="c", subcore_axis_name="s",
        num_cores=NC, num_subcores=NS)

    def body(x_hbm, table_hbm, out_hbm, xblk, rows_v, *sems):
        gsems = sems[:NBUF]
        ssems = sems[NBUF:]
        wid = lax.axis_index("s") * NC + lax.axis_index("c")
        r0 = wid * rows_per_w

        pltpu.sync_copy(x_hbm.at[pl.ds(r0, rows_per_w)], xblk)

        def gather(c, buf):
            return pltpu.make_async_copy(
                table_hbm.at[xblk.at[c]], rows_v.at[buf], gsems[buf])

        def scatter(c, buf):
            return pltpu.make_async_copy(
                rows_v.at[buf], out_hbm.at[r0 + c], ssems[buf])

        gather(0, 0).start()
        gather(1, 1).start()

        @pl.loop(0, rows_per_w, step=NBUF)
        def _group(g):
            for bi in range(NBUF):
                c = g + bi
                if bi >= 2:
                    scatter(c - 2, (bi + 2) % NBUF).wait()
                else:
                    @pl.when(c >= 2)
                    def _():
                        scatter(c - 2, (bi + 2) % NBUF).wait()
                @pl.when(c + 2 < rows_per_w)
                def _():
                    gather(c + 2, (bi + 2) % NBUF).start()
                gather(c, bi).wait()
                buf = rows_v.at[bi]

                @pl.loop(0, s, unroll=8)
                def _row(j):
                    for k in range(d // L):
                        sl = pl.ds(k * L, L)
                        buf[j, sl] = buf[j, sl] * scale

                scatter(c, bi).start()

        scatter(rows_per_w - 2, (rows_per_w - 2) % NBUF).wait()
        scatter(rows_per_w - 1, (rows_per_w - 1) % NBUF).wait()

    return pl.kernel(
        body,
        out_type=jax.ShapeDtypeStruct((b, s, d), jnp.float32),
        mesh=mesh,
        scratch_types=[
            pltpu.VMEM((rows_per_w, s), jnp.int32),
            pltpu.VMEM((NBUF, s, d), jnp.float32),
        ] + [pltpu.SemaphoreType.DMA] * (2 * NBUF),
        compiler_params=pltpu.CompilerParams(use_tc_tiling_on_sc=False),
    )


def kernel(x, table):
    b, s = x.shape
    vocab, d = table.shape
    return _build(b, s, vocab, d)(x.astype(jnp.int32), table)
